# baseline (speedup 1.0000x reference)
import functools

import jax
import jax.numpy as jnp
import numpy as np
from jax import lax
from jax.experimental import pallas as pl
from jax.experimental.pallas import tpu as pltpu
from jax.experimental.pallas import tpu_sc as plsc

B = 16
N = 1024
K = 20
KS = 24
NEG = np.float32(-np.inf)


def _gather(table, idx2d):
    cp = table.shape[-1]
    nrows = table.shape[0]
    nw = 32
    rows_per_w = nrows // nw
    R = 4
    n_chunks = rows_per_w // R

    mesh = plsc.VectorSubcoreMesh(core_axis_name="c", subcore_axis_name="s")

    @functools.partial(
        pl.kernel,
        out_type=jax.ShapeDtypeStruct((nrows * KS, cp), jnp.float32),
        mesh=mesh,
        scratch_types=[
            pltpu.VMEM((n_chunks, R * KS), jnp.int32),
            pltpu.VMEM((2, R * KS, cp), jnp.float32),
            pltpu.SemaphoreType.DMA,
            pltpu.SemaphoreType.DMA,
            pltpu.SemaphoreType.DMA,
            pltpu.SemaphoreType.DMA,
        ],
    )
    def k(table_hbm, idx_hbm, out_hbm, idx_v, rows_v, gs0, gs1, os0, os1):
        wid = lax.axis_index("s") * 2 + lax.axis_index("c")
        base_row = wid * rows_per_w
        gsems = (gs0, gs1)
        osems = (os0, os1)

        pltpu.sync_copy(idx_hbm.at[pl.ds(wid * n_chunks, n_chunks)], idx_v)

        def fire(p, c):
            pltpu.async_copy(table_hbm.at[idx_v.at[c]], rows_v.at[p],
                             gsems[p])

        def consume(p, c):
            r0 = base_row + c * R
            pltpu.make_async_copy(table_hbm.at[idx_v.at[c]], rows_v.at[p],
                                  gsems[p]).wait()
            pltpu.async_copy(rows_v.at[p],
                             out_hbm.at[pl.ds(r0 * KS, R * KS)], osems[p])

        def drain(p):
            pltpu.make_async_copy(
                rows_v.at[p], out_hbm.at[pl.ds(0, R * KS)], osems[p]).wait()

        fire(0, 0)

        def super_step(si, _):
            c0 = si * 2
            fire(1, c0 + 1)
            consume(0, c0)

            @pl.when(c0 + 2 < n_chunks)
            def _():
                fire(0, c0 + 2)

            consume(1, c0 + 1)
            drain(0)
            drain(1)
            return 0

        lax.fori_loop(0, n_chunks // 2, super_step, 0)

    return k(table, idx2d)


def _gather_max(table, idx2d):
    co = table.shape[-1]
    nrows = table.shape[0]
    nw = 32
    rows_per_w = nrows // nw
    R = 4
    n_chunks = rows_per_w // R
    ncc = co // 16

    mesh = plsc.VectorSubcoreMesh(core_axis_name="c", subcore_axis_name="s")

    @functools.partial(
        pl.kernel,
        out_type=jax.ShapeDtypeStruct((nrows, co), jnp.float32),
        mesh=mesh,
        scratch_types=[
            pltpu.VMEM((n_chunks, R * KS), jnp.int32),
            pltpu.VMEM((2, R * KS, co), jnp.float32),
            pltpu.VMEM((2, R, co), jnp.float32),
            pltpu.SemaphoreType.DMA,
            pltpu.SemaphoreType.DMA,
            pltpu.SemaphoreType.DMA,
            pltpu.SemaphoreType.DMA,
        ],
    )
    def k(table_hbm, idx_hbm, out_hbm, idx_v, rows_v, out_v,
          gs0, gs1, os0, os1):
        wid = lax.axis_index("s") * 2 + lax.axis_index("c")
        base_row = wid * rows_per_w
        gsems = (gs0, gs1)
        osems = (os0, os1)

        pltpu.sync_copy(idx_hbm.at[pl.ds(wid * n_chunks, n_chunks)], idx_v)

        def fire(p, c):
            pltpu.async_copy(table_hbm.at[idx_v.at[c]], rows_v.at[p],
                             gsems[p])

        def consume(p, c):
            r0 = base_row + c * R
            pltpu.make_async_copy(table_hbm.at[idx_v.at[c]], rows_v.at[p],
                                  gsems[p]).wait()
            for r in range(R):
                for cc in range(ncc):
                    acc = rows_v[p, r * KS, pl.ds(cc * 16, 16)]
                    for j in range(1, K):
                        acc = jnp.maximum(
                            acc, rows_v[p, r * KS + j, pl.ds(cc * 16, 16)]
                        )
                    out_v[p, r, pl.ds(cc * 16, 16)] = acc
            pltpu.async_copy(out_v.at[p], out_hbm.at[pl.ds(r0, R)], osems[p])

        fire(0, 0)

        def super_step(si, _):
            c0 = si * 2
            fire(1, c0 + 1)
            consume(0, c0)

            @pl.when(c0 + 2 < n_chunks)
            def _():
                fire(0, c0 + 2)

            consume(1, c0 + 1)
            pltpu.make_async_copy(out_v.at[0],
                                  out_hbm.at[pl.ds(base_row, R)], osems[0]
                                  ).wait()
            pltpu.make_async_copy(out_v.at[1],
                                  out_hbm.at[pl.ds(base_row, R)], osems[1]
                                  ).wait()
            return 0

        lax.fori_loop(0, n_chunks // 2, super_step, 0)

    return k(table, idx2d)


def _uv_body(x_ref, wa_ref, wba_ref, bp_ref, u_ref, v_ref):
    x = x_ref[0]
    u_ref[0] = jnp.dot(x, wa_ref[...], preferred_element_type=jnp.float32,
                       precision=lax.Precision.HIGHEST)
    v_ref[0] = (
        jnp.dot(x, wba_ref[...], preferred_element_type=jnp.float32,
                precision=lax.Precision.HIGHEST)
        + bp_ref[...]
    )


def _uv(x, wa, wba, bp):
    ci, co = wa.shape
    nb = x.shape[0]
    return pl.pallas_call(
        _uv_body,
        grid=(nb,),
        in_specs=[
            pl.BlockSpec((1, N, ci), lambda b: (b, 0, 0)),
            pl.BlockSpec((ci, co), lambda b: (0, 0)),
            pl.BlockSpec((ci, co), lambda b: (0, 0)),
            pl.BlockSpec((1, co), lambda b: (0, 0)),
        ],
        out_specs=[
            pl.BlockSpec((1, N, co), lambda b: (b, 0, 0)),
            pl.BlockSpec((1, N, co), lambda b: (b, 0, 0)),
        ],
        out_shape=[
            jax.ShapeDtypeStruct((nb, N, co), jnp.float32),
            jax.ShapeDtypeStruct((nb, N, co), jnp.float32),
        ],
    )(x, wa, wba, bp)


def _econv_body(C, co, cpad, f_ref, c_ref, w_ref, b_ref, g_ref, e_ref,
                x_ref):
    m = c_ref.shape[1]
    f = f_ref[0]
    f3 = f.reshape(m, KS, f.shape[-1])
    c = c_ref[0]
    c3 = c.reshape(m, 1, c.shape[-1])
    fc = f3[:, :, :C] - c3[:, :, :C]
    cb = jnp.broadcast_to(c3[:, :, :C], (m, KS, C))
    parts = [fc, cb]
    if cpad:
        parts.append(jnp.zeros((m, KS, cpad), jnp.float32))
    p = jnp.concatenate(parts, axis=2).reshape(m * KS, 2 * C + cpad)
    y = jnp.dot(p, w_ref[...], preferred_element_type=jnp.float32)
    y = (y + b_ref[...]) * g_ref[...] + e_ref[...]
    y3 = y.reshape(m, KS, co)
    ym = jnp.max(y3[:, :K, :], axis=1)
    x = jnp.maximum(ym, 0.2 * ym)
    if co < 128:
        x = jnp.concatenate([x, jnp.zeros((m, 128 - co), jnp.float32)],
                            axis=1)
    x_ref[0] = x


def _econv(g, xprev, W, bv, gv, ev, C, co, S):
    nb = xprev.shape[0]
    cp = xprev.shape[-1]
    m = N // S
    cpad = (-2 * C) % 8
    wt = W.T
    if cpad:
        wt = jnp.concatenate([wt, jnp.zeros((cpad, co), jnp.float32)], 0)
    co_out = max(co, 128)
    fview = g.reshape(nb * S, m * KS, cp)
    cview = xprev.reshape(nb * S, m, cp)
    return pl.pallas_call(
        functools.partial(_econv_body, C, co, cpad),
        grid=(nb * S,),
        in_specs=[
            pl.BlockSpec((1, m * KS, cp), lambda b: (b, 0, 0)),
            pl.BlockSpec((1, m, cp), lambda b: (b, 0, 0)),
            pl.BlockSpec((2 * C + cpad, co), lambda b: (0, 0)),
            pl.BlockSpec((1, co), lambda b: (0, 0)),
            pl.BlockSpec((1, co), lambda b: (0, 0)),
            pl.BlockSpec((1, co), lambda b: (0, 0)),
        ],
        out_specs=pl.BlockSpec((1, m, co_out), lambda b: (b, 0, 0)),
        out_shape=jax.ShapeDtypeStruct((nb * S, m, co_out), jnp.float32),
    )(fview, cview, wt, bv[None, :], gv[None, :], ev[None, :]).reshape(
        nb, N, co_out)


def _knn_body(x_ref, idx_ref, d_ref):
    x = x_ref[0]
    xx = jnp.sum(x * x, axis=1, keepdims=True)
    inner = lax.dot_general(
        x, x, (((1,), (1,)), ((), ())), preferred_element_type=jnp.float32
    )
    d_ref[...] = -((xx - 2.0 * inner) + jnp.reshape(xx, (1, N)))

    lane = lax.broadcasted_iota(jnp.int32, (N, N), 1)
    lanek = lax.broadcasted_iota(jnp.int32, (N, KS), 1)
    n_i32 = np.int32(N)

    def step(j, carry):
        pv, pi, acc = carry
        d = d_ref[...]
        elig = (d < pv) | ((d == pv) & (lane > pi))
        rm = jnp.max(jnp.where(elig, d, NEG), axis=1, keepdims=True)
        pi_eff = jnp.where(rm == pv, pi, -1)
        msk = (d == rm) & (lane > pi_eff)
        ii = jnp.min(jnp.where(msk, lane, n_i32), axis=1, keepdims=True)
        acc = jnp.where(lanek == j, ii, acc)
        return rm, ii, acc

    pv0 = jnp.full((N, 1), jnp.inf, jnp.float32)
    pi0 = jnp.full((N, 1), -1, jnp.int32)
    acc0 = jnp.zeros((N, KS), jnp.int32)
    _, _, acc = lax.fori_loop(0, K, step, (pv0, pi0, acc0))
    acc = jnp.where(lanek >= K, acc[:, 0:1], acc)
    idx_ref[0] = acc + pl.program_id(0) * N


def _knn(x):
    nb = x.shape[0]
    cp = x.shape[-1]
    return pl.pallas_call(
        _knn_body,
        grid=(nb,),
        in_specs=[pl.BlockSpec((1, N, cp), lambda b: (b, 0, 0))],
        out_specs=pl.BlockSpec((1, N, KS), lambda b: (b, 0, 0)),
        out_shape=jax.ShapeDtypeStruct((nb, N, KS), jnp.int32),
        scratch_shapes=[pltpu.VMEM((N, N), jnp.float32)],
    )(x)


def _head1_body(m_ref, v_ref, x1_ref, x2_ref, x3_ref, wf_ref, bf_ref, o_ref):
    x4 = m_ref[0] + v_ref[0]
    x4 = jnp.maximum(x4, 0.2 * x4)
    xcat = jnp.concatenate(
        [x1_ref[0][:, :64], x2_ref[0][:, :64], x3_ref[0], x4], axis=1)
    y = jnp.dot(xcat, wf_ref[...], preferred_element_type=jnp.float32)
    y = y + bf_ref[...]
    y = jnp.maximum(y, 0.2 * y)
    mx = jnp.max(y, axis=0, keepdims=True)
    mn = jnp.sum(y, axis=0, keepdims=True) * np.float32(1.0 / N)
    o_ref[0] = jnp.concatenate([mx, mn], axis=1)


def _head1(m4, v4, x1, x2, x3, wf_t, bfp):
    nb = m4.shape[0]
    return pl.pallas_call(
        _head1_body,
        grid=(nb,),
        in_specs=[
            pl.BlockSpec((1, N, 256), lambda b: (b, 0, 0)),
            pl.BlockSpec((1, N, 256), lambda b: (b, 0, 0)),
            pl.BlockSpec((1, N, 128), lambda b: (b, 0, 0)),
            pl.BlockSpec((1, N, 128), lambda b: (b, 0, 0)),
            pl.BlockSpec((1, N, 128), lambda b: (b, 0, 0)),
            pl.BlockSpec((512, 1024), lambda b: (0, 0)),
            pl.BlockSpec((1, 1024), lambda b: (0, 0)),
        ],
        out_specs=pl.BlockSpec((1, 1, 2048), lambda b: (b, 0, 0)),
        out_shape=jax.ShapeDtypeStruct((nb, 1, 2048), jnp.float32),
    )(m4, v4, x1, x2, x3, wf_t, bfp)


def _head2_body(p_ref, w1_ref, b1_ref, w2_ref, b2_ref, wo_ref, bo_ref, o_ref):
    h = jnp.dot(p_ref[...], w1_ref[...], preferred_element_type=jnp.float32)
    h = h + b1_ref[...]
    h = jnp.maximum(h, 0.2 * h)
    h = jnp.dot(h, w2_ref[...], preferred_element_type=jnp.float32)
    h = h + b2_ref[...]
    h = jnp.maximum(h, 0.2 * h)
    o_ref[...] = (
        jnp.dot(h, wo_ref[...], preferred_element_type=jnp.float32)
        + bo_ref[...]
    )


def _head2(pooled, w1t, b1p, w2t, b2p, wot, bop):
    return pl.pallas_call(
        _head2_body,
        in_specs=[pl.BlockSpec(a.shape, lambda: tuple(0 for _ in a.shape))
                  for a in (pooled, w1t, b1p, w2t, b2p, wot, bop)],
        out_specs=pl.BlockSpec((B, 128), lambda: (0, 0)),
        out_shape=jax.ShapeDtypeStruct((B, 128), jnp.float32),
    )(pooled, w1t, b1p, w2t, b2p, wot, bop)


def kernel(cloud, indices, W1, b1, g1, e1, W2, b2, g2, e2, W3, b3, g3, e3,
           W4, b4, g4, e4, Wf, bf, gf, ef, Wm1, bm1, gm1, em1,
           Wm2, bm2, gm2, em2, Wo, bo):
    cloudp = jnp.concatenate(
        [cloud, jnp.zeros((B, N, 125), jnp.float32)], axis=-1)

    idx1 = indices.astype(jnp.int32)
    idx1 = jnp.concatenate(
        [idx1, jnp.broadcast_to(idx1[..., :1], (B, N, KS - K))], axis=-1)
    idx1 = idx1 + ((jnp.arange(B, dtype=jnp.int32) % (B // 2)) * N)[
        :, None, None]

    wf_t = (Wf * gf[:, None]).T
    bfp = (bf * gf + ef)[None, :]

    W4p = W4 * g4[:, None]
    wa4 = W4p[:, :128].T
    wba4 = (W4p[:, 128:] - W4p[:, :128]).T
    bp4 = (b4 * g4 + e4)[None, :]

    HB = B // 2
    nix = HB * N // 4
    pooled_halves = []
    for h in range(2):
        sl = slice(h * HB, (h + 1) * HB)
        cp = cloudp[sl]
        g1r = _gather(cp.reshape(HB * N, 128), idx1[sl].reshape(nix, -1))
        x1 = _econv(g1r, cp, W1, b1, g1, e1, C=3, co=64, S=4)

        idx2 = _knn(x1)
        g2r = _gather(x1.reshape(HB * N, 128), idx2.reshape(nix, -1))
        x2 = _econv(g2r, x1, W2, b2, g2, e2, C=64, co=64, S=4)

        idx3 = _knn(x2)
        g3r = _gather(x2.reshape(HB * N, 128), idx3.reshape(nix, -1))
        x3 = _econv(g3r, x2, W3, b3, g3, e3, C=64, co=128, S=4)

        idx4 = _knn(x3)
        u4, v4 = _uv(x3, wa4, wba4, bp4)
        u4f = u4.reshape(HB * N, 256)
        idxr = idx4.reshape(nix, -1)
        m4 = jnp.concatenate(
            [_gather_max(u4f[:, :128], idxr),
             _gather_max(u4f[:, 128:], idxr)], axis=1).reshape(HB, N, 256)

        pooled_halves.append(
            _head1(m4, v4, x1, x2, x3, wf_t, bfp).reshape(HB, 2048))
    pooled = jnp.concatenate(pooled_halves, axis=0)

    w1t = (Wm1 * gm1[:, None]).T
    b1p = (bm1 * gm1 + em1)[None, :]
    w2t = (Wm2 * gm2[:, None]).T
    b2p = (bm2 * gm2 + em2)[None, :]
    wot = jnp.concatenate([Wo.T, jnp.zeros((256, 88), jnp.float32)], axis=1)
    bop = jnp.concatenate([bo, jnp.zeros((88,), jnp.float32)])[None, :]
    out = _head2(pooled, w1t, b1p, w2t, b2p, wot, bop)
    return out[:, :40]


# speedup vs baseline: 1.0141x; 1.0141x over previous
import functools

import jax
import jax.numpy as jnp
import numpy as np
from jax import lax
from jax.experimental import pallas as pl
from jax.experimental.pallas import tpu as pltpu
from jax.experimental.pallas import tpu_sc as plsc

B = 16
N = 1024
K = 20
KS = 24
NEG = np.float32(-np.inf)


def _gather(table, idx2d):
    cp = table.shape[-1]
    nrows = table.shape[0]
    nw = 32
    rows_per_w = nrows // nw
    R = 4
    n_chunks = rows_per_w // R

    mesh = plsc.VectorSubcoreMesh(core_axis_name="c", subcore_axis_name="s")

    @functools.partial(
        pl.kernel,
        out_type=jax.ShapeDtypeStruct((nrows * KS, cp), jnp.float32),
        mesh=mesh,
        scratch_types=[
            pltpu.VMEM((n_chunks, R * KS), jnp.int32),
            pltpu.VMEM((2, R * KS, cp), jnp.float32),
            pltpu.SemaphoreType.DMA,
            pltpu.SemaphoreType.DMA,
            pltpu.SemaphoreType.DMA,
            pltpu.SemaphoreType.DMA,
        ],
    )
    def k(table_hbm, idx_hbm, out_hbm, idx_v, rows_v, gs0, gs1, os0, os1):
        wid = lax.axis_index("s") * 2 + lax.axis_index("c")
        base_row = wid * rows_per_w
        gsems = (gs0, gs1)
        osems = (os0, os1)

        pltpu.sync_copy(idx_hbm.at[pl.ds(wid * n_chunks, n_chunks)], idx_v)

        def fire(p, c):
            pltpu.async_copy(table_hbm.at[idx_v.at[c]], rows_v.at[p],
                             gsems[p])

        def consume(p, c):
            r0 = base_row + c * R
            pltpu.make_async_copy(table_hbm.at[idx_v.at[c]], rows_v.at[p],
                                  gsems[p]).wait()
            pltpu.async_copy(rows_v.at[p],
                             out_hbm.at[pl.ds(r0 * KS, R * KS)], osems[p])

        def drain(p):
            pltpu.make_async_copy(
                rows_v.at[p], out_hbm.at[pl.ds(0, R * KS)], osems[p]).wait()

        fire(0, 0)

        def super_step(si, _):
            c0 = si * 2
            fire(1, c0 + 1)
            consume(0, c0)

            @pl.when(c0 + 2 < n_chunks)
            def _():
                fire(0, c0 + 2)

            consume(1, c0 + 1)
            drain(0)
            drain(1)
            return 0

        lax.fori_loop(0, n_chunks // 2, super_step, 0)

    return k(table, idx2d)


def _econv_body(C, co, cpad, f_ref, c_ref, w_ref, b_ref, g_ref, e_ref,
                x_ref):
    m = c_ref.shape[1]
    f = f_ref[0]
    f3 = f.reshape(m, KS, f.shape[-1])
    c = c_ref[0]
    c3 = c.reshape(m, 1, c.shape[-1])
    fc = f3[:, :, :C] - c3[:, :, :C]
    cb = jnp.broadcast_to(c3[:, :, :C], (m, KS, C))
    parts = [fc, cb]
    if cpad:
        parts.append(jnp.zeros((m, KS, cpad), jnp.float32))
    p = jnp.concatenate(parts, axis=2).reshape(m * KS, 2 * C + cpad)
    y = jnp.dot(p, w_ref[...], preferred_element_type=jnp.float32)
    y = (y + b_ref[...]) * g_ref[...] + e_ref[...]
    y3 = y.reshape(m, KS, co)
    ym = jnp.max(y3[:, :K, :], axis=1)
    x = jnp.maximum(ym, 0.2 * ym)
    if co < 128:
        x = jnp.concatenate([x, jnp.zeros((m, 128 - co), jnp.float32)],
                            axis=1)
    x_ref[0] = x


def _econv(g, xprev, W, bv, gv, ev, C, co, S):
    nb = xprev.shape[0]
    cp = xprev.shape[-1]
    m = N // S
    cpad = (-2 * C) % 8
    wt = W.T
    if cpad:
        wt = jnp.concatenate([wt, jnp.zeros((cpad, co), jnp.float32)], 0)
    co_out = max(co, 128)
    fview = g.reshape(nb * S, m * KS, cp)
    cview = xprev.reshape(nb * S, m, cp)
    return pl.pallas_call(
        functools.partial(_econv_body, C, co, cpad),
        grid=(nb * S,),
        in_specs=[
            pl.BlockSpec((1, m * KS, cp), lambda b: (b, 0, 0)),
            pl.BlockSpec((1, m, cp), lambda b: (b, 0, 0)),
            pl.BlockSpec((2 * C + cpad, co), lambda b: (0, 0)),
            pl.BlockSpec((1, co), lambda b: (0, 0)),
            pl.BlockSpec((1, co), lambda b: (0, 0)),
            pl.BlockSpec((1, co), lambda b: (0, 0)),
        ],
        out_specs=pl.BlockSpec((1, m, co_out), lambda b: (b, 0, 0)),
        out_shape=jax.ShapeDtypeStruct((nb * S, m, co_out), jnp.float32),
    )(fview, cview, wt, bv[None, :], gv[None, :], ev[None, :]).reshape(
        nb, N, co_out)


def _knn_body(x_ref, idx_ref, d_ref):
    x = x_ref[0]
    xx = jnp.sum(x * x, axis=1, keepdims=True)
    inner = lax.dot_general(
        x, x, (((1,), (1,)), ((), ())), preferred_element_type=jnp.float32
    )
    d_ref[...] = -((xx - 2.0 * inner) + jnp.reshape(xx, (1, N)))

    lane = lax.broadcasted_iota(jnp.int32, (N, N), 1)
    lanek = lax.broadcasted_iota(jnp.int32, (N, KS), 1)
    n_i32 = np.int32(N)

    def step(j, carry):
        pv, pi, acc = carry
        d = d_ref[...]
        elig = (d < pv) | ((d == pv) & (lane > pi))
        rm = jnp.max(jnp.where(elig, d, NEG), axis=1, keepdims=True)
        pi_eff = jnp.where(rm == pv, pi, -1)
        msk = (d == rm) & (lane > pi_eff)
        ii = jnp.min(jnp.where(msk, lane, n_i32), axis=1, keepdims=True)
        acc = jnp.where(lanek == j, ii, acc)
        return rm, ii, acc

    pv0 = jnp.full((N, 1), jnp.inf, jnp.float32)
    pi0 = jnp.full((N, 1), -1, jnp.int32)
    acc0 = jnp.zeros((N, KS), jnp.int32)
    _, _, acc = lax.fori_loop(0, K, step, (pv0, pi0, acc0))
    acc = jnp.where(lanek >= K, acc[:, 0:1], acc)
    idx_ref[0] = acc + pl.program_id(0) * N


def _knn(x):
    nb = x.shape[0]
    cp = x.shape[-1]
    return pl.pallas_call(
        _knn_body,
        grid=(nb,),
        in_specs=[pl.BlockSpec((1, N, cp), lambda b: (b, 0, 0))],
        out_specs=pl.BlockSpec((1, N, KS), lambda b: (b, 0, 0)),
        out_shape=jax.ShapeDtypeStruct((nb, N, KS), jnp.int32),
        scratch_shapes=[pltpu.VMEM((N, N), jnp.float32)],
    )(x)


def _head1_body(x4_ref, x1_ref, x2_ref, x3_ref, wf_ref, bf_ref, o_ref):
    xcat = jnp.concatenate(
        [x1_ref[0][:, :64], x2_ref[0][:, :64], x3_ref[0], x4_ref[0]], axis=1)
    y = jnp.dot(xcat, wf_ref[...], preferred_element_type=jnp.float32)
    y = y + bf_ref[...]
    y = jnp.maximum(y, 0.2 * y)
    mx = jnp.max(y, axis=0, keepdims=True)
    mn = jnp.sum(y, axis=0, keepdims=True) * np.float32(1.0 / N)
    o_ref[0] = jnp.concatenate([mx, mn], axis=1)


def _head1(x4, x1, x2, x3, wf_t, bfp):
    nb = x4.shape[0]
    return pl.pallas_call(
        _head1_body,
        grid=(nb,),
        in_specs=[
            pl.BlockSpec((1, N, 256), lambda b: (b, 0, 0)),
            pl.BlockSpec((1, N, 128), lambda b: (b, 0, 0)),
            pl.BlockSpec((1, N, 128), lambda b: (b, 0, 0)),
            pl.BlockSpec((1, N, 128), lambda b: (b, 0, 0)),
            pl.BlockSpec((512, 1024), lambda b: (0, 0)),
            pl.BlockSpec((1, 1024), lambda b: (0, 0)),
        ],
        out_specs=pl.BlockSpec((1, 1, 2048), lambda b: (b, 0, 0)),
        out_shape=jax.ShapeDtypeStruct((nb, 1, 2048), jnp.float32),
    )(x4, x1, x2, x3, wf_t, bfp)


def _head2_body(p_ref, w1_ref, b1_ref, w2_ref, b2_ref, wo_ref, bo_ref, o_ref):
    h = jnp.dot(p_ref[...], w1_ref[...], preferred_element_type=jnp.float32)
    h = h + b1_ref[...]
    h = jnp.maximum(h, 0.2 * h)
    h = jnp.dot(h, w2_ref[...], preferred_element_type=jnp.float32)
    h = h + b2_ref[...]
    h = jnp.maximum(h, 0.2 * h)
    o_ref[...] = (
        jnp.dot(h, wo_ref[...], preferred_element_type=jnp.float32)
        + bo_ref[...]
    )


def _head2(pooled, w1t, b1p, w2t, b2p, wot, bop):
    return pl.pallas_call(
        _head2_body,
        in_specs=[pl.BlockSpec(a.shape, lambda: tuple(0 for _ in a.shape))
                  for a in (pooled, w1t, b1p, w2t, b2p, wot, bop)],
        out_specs=pl.BlockSpec((B, 128), lambda: (0, 0)),
        out_shape=jax.ShapeDtypeStruct((B, 128), jnp.float32),
    )(pooled, w1t, b1p, w2t, b2p, wot, bop)


def kernel(cloud, indices, W1, b1, g1, e1, W2, b2, g2, e2, W3, b3, g3, e3,
           W4, b4, g4, e4, Wf, bf, gf, ef, Wm1, bm1, gm1, em1,
           Wm2, bm2, gm2, em2, Wo, bo):
    cloudp = jnp.concatenate(
        [cloud, jnp.zeros((B, N, 125), jnp.float32)], axis=-1)

    idx1 = indices.astype(jnp.int32)
    idx1 = jnp.concatenate(
        [idx1, jnp.broadcast_to(idx1[..., :1], (B, N, KS - K))], axis=-1)
    idx1 = idx1 + ((jnp.arange(B, dtype=jnp.int32) % (B // 2)) * N)[
        :, None, None]

    wf_t = (Wf * gf[:, None]).T
    bfp = (bf * gf + ef)[None, :]

    HB = B // 2
    nix = HB * N // 4
    pooled_halves = []
    for h in range(2):
        sl = slice(h * HB, (h + 1) * HB)
        cp = cloudp[sl]
        g1r = _gather(cp.reshape(HB * N, 128), idx1[sl].reshape(nix, -1))
        x1 = _econv(g1r, cp, W1, b1, g1, e1, C=3, co=64, S=4)

        idx2 = _knn(x1)
        g2r = _gather(x1.reshape(HB * N, 128), idx2.reshape(nix, -1))
        x2 = _econv(g2r, x1, W2, b2, g2, e2, C=64, co=64, S=4)

        idx3 = _knn(x2)
        g3r = _gather(x2.reshape(HB * N, 128), idx3.reshape(nix, -1))
        x3 = _econv(g3r, x2, W3, b3, g3, e3, C=64, co=128, S=4)

        idx4 = _knn(x3)
        g4r = _gather(x3.reshape(HB * N, 128), idx4.reshape(nix, -1))
        x4 = _econv(g4r, x3, W4, b4, g4, e4, C=128, co=256, S=4)

        pooled_halves.append(
            _head1(x4, x1, x2, x3, wf_t, bfp).reshape(HB, 2048))
    pooled = jnp.concatenate(pooled_halves, axis=0)

    w1t = (Wm1 * gm1[:, None]).T
    b1p = (bm1 * gm1 + em1)[None, :]
    w2t = (Wm2 * gm2[:, None]).T
    b2p = (bm2 * gm2 + em2)[None, :]
    wot = jnp.concatenate([Wo.T, jnp.zeros((256, 88), jnp.float32)], axis=1)
    bop = jnp.concatenate([bo, jnp.zeros((88,), jnp.float32)])[None, :]
    out = _head2(pooled, w1t, b1p, w2t, b2p, wot, bop)
    return out[:, :40]


# speedup vs baseline: 1.0183x; 1.0041x over previous
import functools

import jax
import jax.numpy as jnp
import numpy as np
from jax import lax
from jax.experimental import pallas as pl
from jax.experimental.pallas import tpu as pltpu
from jax.experimental.pallas import tpu_sc as plsc

B = 16
N = 1024
K = 20
KS = 24
NEG = np.float32(-np.inf)


def _gather(table, idx2d):
    cp = table.shape[-1]
    nrows = table.shape[0]
    nw = 32
    rows_per_w = nrows // nw
    R = 4
    n_chunks = rows_per_w // R

    mesh = plsc.VectorSubcoreMesh(core_axis_name="c", subcore_axis_name="s")

    @functools.partial(
        pl.kernel,
        out_type=jax.ShapeDtypeStruct((nrows * KS, cp), jnp.float32),
        mesh=mesh,
        scratch_types=[
            pltpu.VMEM((n_chunks, R * KS), jnp.int32),
            pltpu.VMEM((2, R * KS, cp), jnp.float32),
            pltpu.SemaphoreType.DMA,
            pltpu.SemaphoreType.DMA,
            pltpu.SemaphoreType.DMA,
            pltpu.SemaphoreType.DMA,
        ],
    )
    def k(table_hbm, idx_hbm, out_hbm, idx_v, rows_v, gs0, gs1, os0, os1):
        wid = lax.axis_index("s") * 2 + lax.axis_index("c")
        base_row = wid * rows_per_w
        gsems = (gs0, gs1)
        osems = (os0, os1)

        pltpu.sync_copy(idx_hbm.at[pl.ds(wid * n_chunks, n_chunks)], idx_v)

        def fire(p, c):
            pltpu.async_copy(table_hbm.at[idx_v.at[c]], rows_v.at[p],
                             gsems[p])

        def consume(p, c):
            r0 = base_row + c * R
            pltpu.make_async_copy(table_hbm.at[idx_v.at[c]], rows_v.at[p],
                                  gsems[p]).wait()
            pltpu.async_copy(rows_v.at[p],
                             out_hbm.at[pl.ds(r0 * KS, R * KS)], osems[p])

        def drain(p):
            pltpu.make_async_copy(
                rows_v.at[p], out_hbm.at[pl.ds(0, R * KS)], osems[p]).wait()

        fire(0, 0)

        def super_step(si, _):
            c0 = si * 2
            fire(1, c0 + 1)
            consume(0, c0)

            @pl.when(c0 + 2 < n_chunks)
            def _():
                fire(0, c0 + 2)

            consume(1, c0 + 1)
            drain(0)
            drain(1)
            return 0

        lax.fori_loop(0, n_chunks // 2, super_step, 0)

    return k(table, idx2d)


def _econv_body(C, co, cpad, f_ref, c_ref, w_ref, b_ref, g_ref, e_ref,
                x_ref):
    m = c_ref.shape[1]
    f = f_ref[0]
    f3 = f.reshape(m, KS, f.shape[-1])
    c = c_ref[0]
    c3 = c.reshape(m, 1, c.shape[-1])
    fc = f3[:, :, :C] - c3[:, :, :C]
    cb = jnp.broadcast_to(c3[:, :, :C], (m, KS, C))
    parts = [fc, cb]
    if cpad:
        parts.append(jnp.zeros((m, KS, cpad), jnp.float32))
    p = jnp.concatenate(parts, axis=2).reshape(m * KS, 2 * C + cpad)
    y = jnp.dot(p, w_ref[...], preferred_element_type=jnp.float32)
    y = (y + b_ref[...]) * g_ref[...] + e_ref[...]
    y3 = y.reshape(m, KS, co)
    ym = jnp.max(y3[:, :K, :], axis=1)
    x = jnp.maximum(ym, 0.2 * ym)
    if co < 128:
        x = jnp.concatenate([x, jnp.zeros((m, 128 - co), jnp.float32)],
                            axis=1)
    x_ref[0] = x


def _econv(g, xprev, W, bv, gv, ev, C, co, S):
    nb = xprev.shape[0]
    cp = xprev.shape[-1]
    m = N // S
    cpad = (-2 * C) % 8
    wt = W.T
    if cpad:
        wt = jnp.concatenate([wt, jnp.zeros((cpad, co), jnp.float32)], 0)
    co_out = max(co, 128)
    fview = g.reshape(nb * S, m * KS, cp)
    cview = xprev.reshape(nb * S, m, cp)
    return pl.pallas_call(
        functools.partial(_econv_body, C, co, cpad),
        grid=(nb * S,),
        in_specs=[
            pl.BlockSpec((1, m * KS, cp), lambda b: (b, 0, 0)),
            pl.BlockSpec((1, m, cp), lambda b: (b, 0, 0)),
            pl.BlockSpec((2 * C + cpad, co), lambda b: (0, 0)),
            pl.BlockSpec((1, co), lambda b: (0, 0)),
            pl.BlockSpec((1, co), lambda b: (0, 0)),
            pl.BlockSpec((1, co), lambda b: (0, 0)),
        ],
        out_specs=pl.BlockSpec((1, m, co_out), lambda b: (b, 0, 0)),
        out_shape=jax.ShapeDtypeStruct((nb * S, m, co_out), jnp.float32),
    )(fview, cview, wt, bv[None, :], gv[None, :], ev[None, :]).reshape(
        nb, N, co_out)


def _knn_body(x_ref, idx_ref, d_ref):
    x = x_ref[0]
    xx = jnp.sum(x * x, axis=1, keepdims=True)
    inner = lax.dot_general(
        x, x, (((1,), (1,)), ((), ())), preferred_element_type=jnp.float32
    )
    d_ref[...] = -((xx - 2.0 * inner) + jnp.reshape(xx, (1, N)))

    lane = lax.broadcasted_iota(jnp.int32, (N, N), 1)
    lanek = lax.broadcasted_iota(jnp.int32, (N, KS), 1)
    n_i32 = np.int32(N)

    def step(j, carry):
        pv, pi, acc = carry
        d = d_ref[...]
        elig = (d < pv) | ((d == pv) & (lane > pi))
        rm = jnp.max(jnp.where(elig, d, NEG), axis=1, keepdims=True)
        pi_eff = jnp.where(rm == pv, pi, -1)
        msk = (d == rm) & (lane > pi_eff)
        ii = jnp.min(jnp.where(msk, lane, n_i32), axis=1, keepdims=True)
        acc = jnp.where(lanek == j, ii, acc)
        return rm, ii, acc

    pv0 = jnp.full((N, 1), jnp.inf, jnp.float32)
    pi0 = jnp.full((N, 1), -1, jnp.int32)
    acc0 = jnp.zeros((N, KS), jnp.int32)
    _, _, acc = lax.fori_loop(0, K, step, (pv0, pi0, acc0))
    acc = jnp.where(lanek >= K, acc[:, 0:1], acc)
    idx_ref[0] = acc + pl.program_id(0) * N


def _knn(x):
    nb = x.shape[0]
    cp = x.shape[-1]
    return pl.pallas_call(
        _knn_body,
        grid=(nb,),
        in_specs=[pl.BlockSpec((1, N, cp), lambda b: (b, 0, 0))],
        out_specs=pl.BlockSpec((1, N, KS), lambda b: (b, 0, 0)),
        out_shape=jax.ShapeDtypeStruct((nb, N, KS), jnp.int32),
        scratch_shapes=[pltpu.VMEM((N, N), jnp.float32)],
    )(x)


def _head1_body(x4_ref, x1_ref, x2_ref, x3_ref, wf_ref, bf_ref, o_ref):
    xcat = jnp.concatenate(
        [x1_ref[0][:, :64], x2_ref[0][:, :64], x3_ref[0], x4_ref[0]], axis=1)
    y = jnp.dot(xcat, wf_ref[...], preferred_element_type=jnp.float32)
    y = y + bf_ref[...]
    y = jnp.maximum(y, 0.2 * y)
    mx = jnp.max(y, axis=0, keepdims=True)
    mn = jnp.sum(y, axis=0, keepdims=True) * np.float32(1.0 / N)
    o_ref[0] = jnp.concatenate([mx, mn], axis=1)


def _head1(x4, x1, x2, x3, wf_t, bfp):
    nb = x4.shape[0]
    return pl.pallas_call(
        _head1_body,
        grid=(nb,),
        in_specs=[
            pl.BlockSpec((1, N, 256), lambda b: (b, 0, 0)),
            pl.BlockSpec((1, N, 128), lambda b: (b, 0, 0)),
            pl.BlockSpec((1, N, 128), lambda b: (b, 0, 0)),
            pl.BlockSpec((1, N, 128), lambda b: (b, 0, 0)),
            pl.BlockSpec((512, 1024), lambda b: (0, 0)),
            pl.BlockSpec((1, 1024), lambda b: (0, 0)),
        ],
        out_specs=pl.BlockSpec((1, 1, 2048), lambda b: (b, 0, 0)),
        out_shape=jax.ShapeDtypeStruct((nb, 1, 2048), jnp.float32),
    )(x4, x1, x2, x3, wf_t, bfp)


def _head2_body(p_ref, w1_ref, b1_ref, w2_ref, b2_ref, wo_ref, bo_ref, o_ref):
    h = jnp.dot(p_ref[...], w1_ref[...], preferred_element_type=jnp.float32)
    h = h + b1_ref[...]
    h = jnp.maximum(h, 0.2 * h)
    h = jnp.dot(h, w2_ref[...], preferred_element_type=jnp.float32)
    h = h + b2_ref[...]
    h = jnp.maximum(h, 0.2 * h)
    o_ref[...] = (
        jnp.dot(h, wo_ref[...], preferred_element_type=jnp.float32)
        + bo_ref[...]
    )


def _head2(pooled, w1t, b1p, w2t, b2p, wot, bop):
    return pl.pallas_call(
        _head2_body,
        in_specs=[pl.BlockSpec(a.shape, lambda: tuple(0 for _ in a.shape))
                  for a in (pooled, w1t, b1p, w2t, b2p, wot, bop)],
        out_specs=pl.BlockSpec((B, 128), lambda: (0, 0)),
        out_shape=jax.ShapeDtypeStruct((B, 128), jnp.float32),
    )(pooled, w1t, b1p, w2t, b2p, wot, bop)


def kernel(cloud, indices, W1, b1, g1, e1, W2, b2, g2, e2, W3, b3, g3, e3,
           W4, b4, g4, e4, Wf, bf, gf, ef, Wm1, bm1, gm1, em1,
           Wm2, bm2, gm2, em2, Wo, bo):
    cloudp = jnp.concatenate(
        [cloud, jnp.zeros((B, N, 125), jnp.float32)], axis=-1)

    idx1 = indices.astype(jnp.int32)
    idx1 = jnp.concatenate(
        [idx1, jnp.broadcast_to(idx1[..., :1], (B, N, KS - K))], axis=-1)
    idx1 = idx1 + ((jnp.arange(B, dtype=jnp.int32) % (B // 4)) * N)[
        :, None, None]

    wf_t = (Wf * gf[:, None]).T
    bfp = (bf * gf + ef)[None, :]

    HB = B // 4
    nix = HB * N // 4
    pooled_halves = []
    for h in range(4):
        sl = slice(h * HB, (h + 1) * HB)
        cp = cloudp[sl]
        g1r = _gather(cp.reshape(HB * N, 128), idx1[sl].reshape(nix, -1))
        x1 = _econv(g1r, cp, W1, b1, g1, e1, C=3, co=64, S=4)

        idx2 = _knn(x1)
        g2r = _gather(x1.reshape(HB * N, 128), idx2.reshape(nix, -1))
        x2 = _econv(g2r, x1, W2, b2, g2, e2, C=64, co=64, S=4)

        idx3 = _knn(x2)
        g3r = _gather(x2.reshape(HB * N, 128), idx3.reshape(nix, -1))
        x3 = _econv(g3r, x2, W3, b3, g3, e3, C=64, co=128, S=4)

        idx4 = _knn(x3)
        g4r = _gather(x3.reshape(HB * N, 128), idx4.reshape(nix, -1))
        x4 = _econv(g4r, x3, W4, b4, g4, e4, C=128, co=256, S=4)

        pooled_halves.append(
            _head1(x4, x1, x2, x3, wf_t, bfp).reshape(HB, 2048))
    pooled = jnp.concatenate(pooled_halves, axis=0)

    w1t = (Wm1 * gm1[:, None]).T
    b1p = (bm1 * gm1 + em1)[None, :]
    w2t = (Wm2 * gm2[:, None]).T
    b2p = (bm2 * gm2 + em2)[None, :]
    wot = jnp.concatenate([Wo.T, jnp.zeros((256, 88), jnp.float32)], axis=1)
    bop = jnp.concatenate([bo, jnp.zeros((88,), jnp.float32)])[None, :]
    out = _head2(pooled, w1t, b1p, w2t, b2p, wot, bop)
    return out[:, :40]
